# baseline (speedup 1.0000x reference)
import functools

import jax
import jax.numpy as jnp
from jax import lax
from jax.experimental import pallas as pl
from jax.experimental.pallas import tpu as pltpu
from jax.experimental.pallas import tpu_sc as plsc

D = 64
LANES = 16
NC = 2
NS = 16
NW = NC * NS
C = 128
SCALE = 8.0


def _build(n_total):
  assert n_total % (NW * C) == 0
  nch = n_total // (NW * C)
  mesh = plsc.VectorSubcoreMesh(core_axis_name="c", subcore_axis_name="s")

  @functools.partial(
      pl.kernel,
      out_type=jax.ShapeDtypeStruct((n_total, D), jnp.float32),
      mesh=mesh,
      scratch_types=[
          pltpu.VMEM((nch, C), jnp.int32),
          pltpu.VMEM((C, D), jnp.float32),
          pltpu.SemaphoreType.DMA,
          pltpu.SemaphoreType.DMA,
      ],
      compiler_params=pltpu.CompilerParams(use_tc_tiling_on_sc=False),
  )
  def emb(x_hbm, table_hbm, out_hbm, idx_v, rows_v, sem_in, sem_out):
    wid = lax.axis_index("s") * NC + lax.axis_index("c")
    row0 = wid * (nch * C)
    pltpu.sync_copy(x_hbm.at[wid], idx_v)

    def chunk(j, carry):
      pltpu.async_copy(table_hbm.at[idx_v.at[j]], rows_v, sem_in).wait()

      def srow(r, c2):
        for c4 in range(D // LANES):
          sl = pl.ds(c4 * LANES, LANES)
          rows_v[r, sl] = rows_v[r, sl] * SCALE
        return c2

      lax.fori_loop(0, C, srow, 0)
      pltpu.async_copy(
          rows_v, out_hbm.at[pl.ds(row0 + j * C, C)], sem_out
      ).wait()
      return carry

    lax.fori_loop(0, nch, chunk, 0)

  return emb


_N_TOTAL = 16384 * 50
_EMB = _build(_N_TOTAL)


def kernel(x, table):
  b, l = x.shape
  xr = x.reshape(NW, _N_TOTAL // (NW * C), C)
  out = _EMB(xr, table)
  return out.reshape(b, l, D)
